# Initial kernel scaffold; baseline (speedup 1.0000x reference)
import jax
import jax.numpy as jnp
from jax.experimental import pallas as pl


def kernel(x, emb_w, W1, b1, Wh, bh, Wo, bo):
    raise NotImplementedError("write your pallas kernel here")


# speedup vs baseline: 3.9221x; 3.9221x over previous
import jax
import jax.numpy as jnp
from jax.experimental import pallas as pl

_B = 1024
_D_IN = 64
_H = 1024
_D_OUT = 64
_K = 1024
_EDIM = 64
_N = _B * _D_IN


def _dist_kernel(xr_ref, emb_ref, ind_ref, qlat_ref):
    j = pl.program_id(0)
    xb = xr_ref[0]
    xbb = jnp.broadcast_to(xb, (_EDIM, 1024))
    e = jax.lax.broadcasted_iota(jnp.int32, (_EDIM, 1024), 0) + 1
    p = xbb
    acc = jnp.ones((_EDIM, 1024), jnp.float32)
    for b in range(7):
        acc = jnp.where(((e >> b) & 1) == 1, acc * p, acc)
        if b < 6:
            p = p * p
    P = acc
    emb = emb_ref[...]
    embsq = jnp.sum(emb * emb, axis=1, keepdims=True)
    sm = jnp.sum(P, axis=0, keepdims=True)
    dist = (sm + embsq) - 2.0 * jnp.dot(
        emb, P, preferred_element_type=jnp.float32)
    minv = jnp.min(dist, axis=0)
    amin = jnp.argmin(dist, axis=0)
    ind_ref[0] = amin.astype(jnp.float32)[None, :]
    rowsq = jnp.sum(P * P, axis=0)
    part = jnp.sum(rowsq + (minv - sm[0])).reshape(1, 1)

    @pl.when(j == 0)
    def _init():
        qlat_ref[...] = jnp.zeros((1, 1), jnp.float32)

    qlat_ref[...] += part


def _mlp_kernel(ind_ref, x_ref, w1t_ref, b1_ref, wht_ref, bh_ref,
                wot_ref, bo_ref, qlat_ref, f_ref, loss_ref):
    q = jnp.transpose(ind_ref[...])
    h = jnp.maximum(
        jnp.dot(q, w1t_ref[...], preferred_element_type=jnp.float32)
        + b1_ref[...], 0.0)
    for _ in range(4):
        h = jnp.maximum(
            jnp.dot(h, wht_ref[...], preferred_element_type=jnp.float32)
            + bh_ref[...], 0.0)
    f_ref[...] = jnp.maximum(
        jnp.dot(h, wot_ref[...], preferred_element_type=jnp.float32)
        + bo_ref[...], 0.0)
    d = x_ref[...] - q
    e_sum = jnp.sum(d * d)
    loss_ref[...] = (qlat_ref[...] * (1.0 / (_N * _EDIM))
                     + 0.25 * e_sum.reshape(1, 1) * (1.0 / _N))


def kernel(x, emb_w, W1, b1, Wh, bh, Wo, bo):
    xr3 = x.reshape(_EDIM, 1, 1024)

    ind_mat, qlat = pl.pallas_call(
        _dist_kernel,
        grid=(_EDIM,),
        in_specs=[
            pl.BlockSpec((1, 1, 1024), lambda j: (j, 0, 0)),
            pl.BlockSpec((_K, _EDIM), lambda j: (0, 0)),
        ],
        out_specs=[
            pl.BlockSpec((1, 1, 1024), lambda j: (j, 0, 0)),
            pl.BlockSpec((1, 1), lambda j: (0, 0)),
        ],
        out_shape=[
            jax.ShapeDtypeStruct((_EDIM, 1, 1024), jnp.float32),
            jax.ShapeDtypeStruct((1, 1), jnp.float32),
        ],
    )(xr3, emb_w)

    f, loss = pl.pallas_call(
        _mlp_kernel,
        out_shape=[
            jax.ShapeDtypeStruct((_B, _D_OUT), jnp.float32),
            jax.ShapeDtypeStruct((1, 1), jnp.float32),
        ],
    )(ind_mat.reshape(_EDIM, 1024), x, W1.T, b1.reshape(1, _H),
      Wh.T, bh.reshape(1, _H), Wo.T, bo.reshape(1, _D_OUT), qlat)

    return f, loss[0, 0]


# speedup vs baseline: 4.3646x; 1.1128x over previous
import jax
import jax.numpy as jnp
from jax.experimental import pallas as pl

_B = 1024
_D_IN = 64
_H = 1024
_D_OUT = 64
_K = 1024
_EDIM = 64
_N = _B * _D_IN


def _dist_kernel(xr_ref, emb_ref, ind_ref, qlat_ref):
    j = pl.program_id(0)
    xb = xr_ref[0]
    xbb = jnp.broadcast_to(xb, (_EDIM, 1024))
    e = jax.lax.broadcasted_iota(jnp.int32, (_EDIM, 1024), 0) + 1
    p = xbb
    acc = jnp.ones((_EDIM, 1024), jnp.float32)
    for b in range(7):
        acc = jnp.where(((e >> b) & 1) == 1, acc * p, acc)
        if b < 6:
            p = p * p
    P = acc
    emb = emb_ref[...]
    embsq = jnp.sum(emb * emb, axis=1, keepdims=True)
    sm = jnp.sum(P, axis=0, keepdims=True)
    dist = (sm + embsq) + jnp.dot(
        emb * -2.0, P, preferred_element_type=jnp.float32)
    vals = dist
    idxs = jax.lax.broadcasted_iota(jnp.int32, (_K, 1024), 0)
    h = _K // 2
    while h >= 8:
        mask = vals[:h] <= vals[h:]
        vals = jnp.minimum(vals[:h], vals[h:])
        idxs = jnp.where(mask, idxs[:h], idxs[h:])
        h //= 2
    minv = jnp.min(vals, axis=0)
    amin = jnp.min(
        jnp.where(vals == minv[None, :], idxs, jnp.int32(1 << 30)),
        axis=0)
    ind_ref[0] = amin.astype(jnp.float32)[None, :]
    rowsq = jnp.sum(P * P, axis=0)
    part = jnp.sum(rowsq + (minv - sm[0])).reshape(1, 1)

    @pl.when(j == 0)
    def _init():
        qlat_ref[...] = jnp.zeros((1, 1), jnp.float32)

    qlat_ref[...] += part


def _nt_dot(a, b):
    return jax.lax.dot_general(a, b, (((1,), (1,)), ((), ())),
                               preferred_element_type=jnp.float32)


def _mlp_kernel(ind_ref, x_ref, w1_ref, b1_ref, wh_ref, bh_ref,
                wo_ref, bo_ref, qlat_ref, f_ref, loss_ref):
    q = jnp.transpose(ind_ref[...])
    h = jnp.maximum(_nt_dot(q, w1_ref[...]) + b1_ref[...], 0.0)
    for _ in range(4):
        h = jnp.maximum(_nt_dot(h, wh_ref[...]) + bh_ref[...], 0.0)
    f_ref[...] = jnp.maximum(_nt_dot(h, wo_ref[...]) + bo_ref[...], 0.0)
    d = x_ref[...] - q
    e_sum = jnp.sum(d * d)
    loss_ref[...] = (qlat_ref[...] * (1.0 / (_N * _EDIM))
                     + 0.25 * e_sum.reshape(1, 1) * (1.0 / _N))


def kernel(x, emb_w, W1, b1, Wh, bh, Wo, bo):
    xr3 = x.reshape(_EDIM, 1, 1024)

    ind_mat, qlat = pl.pallas_call(
        _dist_kernel,
        grid=(_EDIM,),
        in_specs=[
            pl.BlockSpec((1, 1, 1024), lambda j: (j, 0, 0)),
            pl.BlockSpec((_K, _EDIM), lambda j: (0, 0)),
        ],
        out_specs=[
            pl.BlockSpec((1, 1, 1024), lambda j: (j, 0, 0)),
            pl.BlockSpec((1, 1), lambda j: (0, 0)),
        ],
        out_shape=[
            jax.ShapeDtypeStruct((_EDIM, 1, 1024), jnp.float32),
            jax.ShapeDtypeStruct((1, 1), jnp.float32),
        ],
    )(xr3, emb_w)

    f, loss = pl.pallas_call(
        _mlp_kernel,
        out_shape=[
            jax.ShapeDtypeStruct((_B, _D_OUT), jnp.float32),
            jax.ShapeDtypeStruct((1, 1), jnp.float32),
        ],
    )(ind_mat.reshape(_EDIM, 1024), x, W1, b1.reshape(1, _H),
      Wh, bh.reshape(1, _H), Wo, bo.reshape(1, _D_OUT), qlat)

    return f, loss[0, 0]


# speedup vs baseline: 4.4910x; 1.0290x over previous
import jax
import jax.numpy as jnp
from jax.experimental import pallas as pl
from jax.experimental.pallas import tpu as pltpu

_B = 1024
_D_IN = 64
_H = 1024
_D_OUT = 64
_K = 1024
_EDIM = 64
_N = _B * _D_IN


def _nt_dot(a, b):
    return jax.lax.dot_general(a, b, (((1,), (1,)), ((), ())),
                               preferred_element_type=jnp.float32)


def _fused_kernel(xr_ref, emb_ref, x_ref, w1_ref, b1_ref, wh_ref, bh_ref,
                  wo_ref, bo_ref, f_ref, loss_ref, ind_ref, qlat_ref):
    j = pl.program_id(0)

    @pl.when(j < _EDIM)
    def _dist_step():
        xb = xr_ref[0]
        xbb = jnp.broadcast_to(xb, (_EDIM, 1024))
        e = jax.lax.broadcasted_iota(jnp.int32, (_EDIM, 1024), 0) + 1
        p = xbb
        acc = jnp.ones((_EDIM, 1024), jnp.float32)
        for b in range(7):
            acc = jnp.where(((e >> b) & 1) == 1, acc * p, acc)
            if b < 6:
                p = p * p
        P = acc
        emb = emb_ref[...]
        embsq = jnp.sum(emb * emb, axis=1, keepdims=True)
        sm = jnp.sum(P, axis=0, keepdims=True)
        dist = (sm + embsq) + jnp.dot(
            emb * -2.0, P, preferred_element_type=jnp.float32)
        vals = dist
        idxs = jax.lax.broadcasted_iota(jnp.int32, (_K, 1024), 0)
        h = _K // 2
        while h >= 8:
            mask = vals[:h] <= vals[h:]
            vals = jnp.minimum(vals[:h], vals[h:])
            idxs = jnp.where(mask, idxs[:h], idxs[h:])
            h //= 2
        minv = jnp.min(vals, axis=0)
        amin = jnp.min(
            jnp.where(vals == minv[None, :], idxs, jnp.int32(1 << 30)),
            axis=0)
        ind_ref[pl.ds(j, 1), :] = amin.astype(jnp.float32)[None, :]
        rowsq = jnp.sum(P * P, axis=0)
        part = jnp.sum(rowsq + (minv - sm[0]))

        @pl.when(j == 0)
        def _init():
            qlat_ref[0, 0] = 0.0

        qlat_ref[0, 0] += part

    @pl.when(j == _EDIM)
    def _mlp_step():
        q = jnp.transpose(ind_ref[...])
        h = jnp.maximum(_nt_dot(q, w1_ref[...]) + b1_ref[...], 0.0)
        for _ in range(4):
            h = jnp.maximum(_nt_dot(h, wh_ref[...]) + bh_ref[...], 0.0)
        f_ref[...] = jnp.maximum(_nt_dot(h, wo_ref[...]) + bo_ref[...], 0.0)
        d = x_ref[...] - q
        e_sum = jnp.sum(d * d)
        loss_ref[...] = (qlat_ref[0, 0] * (1.0 / (_N * _EDIM))
                         + 0.25 * e_sum * (1.0 / _N)).reshape(1, 1)


def kernel(x, emb_w, W1, b1, Wh, bh, Wo, bo):
    xr3 = x.reshape(_EDIM, 1, 1024)
    last = _EDIM - 1

    const = lambda *blk: pl.BlockSpec(blk, lambda j: tuple(0 for _ in blk))
    f, loss = pl.pallas_call(
        _fused_kernel,
        grid=(_EDIM + 1,),
        in_specs=[
            pl.BlockSpec((1, 1, 1024), lambda j: (jnp.minimum(j, last), 0, 0)),
            const(_K, _EDIM),
            const(_B, _D_IN),
            const(_H, _D_IN),
            const(1, _H),
            const(_H, _H),
            const(1, _H),
            const(_D_OUT, _H),
            const(1, _D_OUT),
        ],
        out_specs=[
            const(_B, _D_OUT),
            const(1, 1),
        ],
        out_shape=[
            jax.ShapeDtypeStruct((_B, _D_OUT), jnp.float32),
            jax.ShapeDtypeStruct((1, 1), jnp.float32),
        ],
        scratch_shapes=[
            pltpu.VMEM((_EDIM, 1024), jnp.float32),
            pltpu.SMEM((1, 1), jnp.float32),
        ],
    )(xr3, emb_w, x, W1, b1.reshape(1, _H), Wh, bh.reshape(1, _H),
      Wo, bo.reshape(1, _D_OUT))

    return f, loss[0, 0]
